# baseline (speedup 1.0000x reference)
import functools

import jax
import jax.numpy as jnp
from jax import lax
from jax.experimental import pallas as pl
from jax.experimental.pallas import tpu as pltpu
from jax.experimental.pallas import tpu_sc as plsc

NC = 2
NS = 16
NW = NC * NS
LANES = 16
VSIZE = 32768


def _mesh():
    return plsc.VectorSubcoreMesh(
        core_axis_name="c", subcore_axis_name="s",
        num_cores=NC, num_subcores=NS)


def _make_fused_kernel(L, D, P):
    nchunk = L // LANES
    b_per_w = L // NW
    K = 16
    nsub = b_per_w // K
    NR = 3
    NE = 2

    @functools.partial(
        pl.kernel,
        out_type=jax.ShapeDtypeStruct((L, D), jnp.float32),
        mesh=_mesh(),
        scratch_types=[
            pltpu.VMEM((VSIZE,), jnp.int32),
            pltpu.VMEM((L,), jnp.int32),
            pltpu.VMEM((L,), jnp.int32),
            pltpu.VMEM((L,), jnp.int32),
            pltpu.VMEM((L // NS,), jnp.int32),
            pltpu.VMEM((LANES,), jnp.int32),
            pltpu.VMEM((b_per_w,), jnp.int32),
            pltpu.VMEM((NR, K, D), jnp.float32),
            pltpu.VMEM((NE, K, D), jnp.float32),
            pltpu.VMEM_SHARED((L,), jnp.int32),
            pltpu.VMEM_SHARED((L,), jnp.int32),
            pltpu.SemaphoreType.DMA,
            pltpu.SemaphoreType.DMA,
            pltpu.SemaphoreType.DMA,
        ],
        compiler_params=pltpu.CompilerParams(needs_layout_passes=False),
    )
    def fused(tok_hbm, emb_hbm, pe_hbm, out_hbm,
              table, tok_v, pos_v, firstv_v, fbuf, shuf, idx_v,
              rows_v, emb_v, firstv_sh, pos_sh, gsem, esem, osem):
        sid = lax.axis_index("s")
        cid = lax.axis_index("c")
        wid = sid * NC + cid
        base = wid * b_per_w
        CPT = nchunk // NS
        TPT = CPT * LANES

        ce = [None] * nsub

        def issue_emb(j):
            ce[j] = pltpu.async_copy(
                emb_hbm.at[pl.ds(base + j * K, K)], emb_v.at[j % NE], esem)

        issue_emb(0)
        if nsub > 1:
            issue_emb(1)

        lane = lax.iota(jnp.int32, LANES)
        prev = jnp.maximum(lane - 1, 0)
        tstart = sid * TPT

        @pl.when(sid == 0)
        def _():
            pltpu.sync_copy(tok_hbm, tok_v)

        @pl.when(sid != 0)
        def _():
            pltpu.sync_copy(tok_hbm.at[pl.ds(tstart, TPT)],
                            tok_v.at[pl.ds(tstart, TPT)])

        def dedup_body(k, _):
            t = tok_v[pl.ds(tstart + k * LANES, LANES)]
            sk, sv = plsc.sort_key_val(t * LANES + lane, lane)
            shuf[...] = sk
            sp = plsc.load_gather(shuf, [prev])
            bound = ((sk >> 4) != (sp >> 4)) | (lane == 0)
            plsc.store_scatter(fbuf, [k * LANES + sv], bound.astype(jnp.int32))
            return 0

        with jax.named_scope("p1a_dedup"):
            lax.fori_loop(0, CPT, dedup_body, 0)
            pltpu.sync_copy(fbuf, firstv_sh.at[pl.ds(tstart, TPT)])

        @pl.when(sid == 0)
        def _():
            neg1 = jnp.full((LANES,), -1, jnp.int32)

            def init_body(i, _):
                table[pl.ds(i * LANES, LANES)] = neg1
                return 0

            lax.fori_loop(0, VSIZE // LANES, init_body, 0)

        plsc.subcore_barrier()

        @pl.when(sid == 0)
        def _():
          with jax.named_scope("p1b_rank"):
            pltpu.sync_copy(firstv_sh, firstv_v)

            def chunk_body(ci, counter):
                t = tok_v[pl.ds(ci * LANES, LANES)]
                g = plsc.load_gather(table, [t])
                firstv = firstv_v[pl.ds(ci * LANES, LANES)]
                new = (g < 0) & (firstv > 0)
                newi = new.astype(jnp.int32)
                inc = plsc.cumsum(newi)
                rank_new = counter + inc - newi
                plsc.store_scatter(table, [t], rank_new, mask=new)
                return counter + plsc.all_reduce_population_count(new)

            lax.fori_loop(0, nchunk, chunk_body, jnp.zeros((LANES,), jnp.int32))

            def pos_body(ci, _):
                t = tok_v[pl.ds(ci * LANES, LANES)]
                pos_v[pl.ds(ci * LANES, LANES)] = plsc.load_gather(table, [t])
                return 0

            lax.fori_loop(0, nchunk, pos_body, 0)
            pltpu.sync_copy(pos_v, pos_sh)

        plsc.subcore_barrier()

        scope2 = jax.named_scope("p2_gather_add")
        scope2.__enter__()
        pltpu.sync_copy(pos_sh.at[pl.ds(base, b_per_w)], idx_v)
        cg = [None] * nsub
        co = [None] * nsub

        def issue_gather(j):
            cg[j] = pltpu.async_copy(
                pe_hbm.at[idx_v.at[pl.ds(j * K, K)]], rows_v.at[j % NR], gsem)

        issue_gather(0)
        if nsub > 1:
            issue_gather(1)
        for j in range(nsub):
            if j + 1 < nsub and j >= 1:
                issue_emb(j + 1)
            if j + 2 < nsub:
                if j + 2 >= NR:
                    co[j - 1].wait()
                issue_gather(j + 2)
            cg[j].wait()
            ce[j].wait()
            rs, es = j % NR, j % NE

            def add_body(r, _):
                for cc in range(D // LANES):
                    sl = pl.ds(cc * LANES, LANES)
                    rows_v[rs, r, sl] += emb_v[es, r, sl]
                return 0

            lax.fori_loop(0, K, add_body, 0)
            co[j] = pltpu.async_copy(
                rows_v.at[rs], out_hbm.at[pl.ds(base + j * K, K)], osem)
        for j in range(max(nsub - NR, 0), nsub):
            co[j].wait()
        scope2.__exit__(None, None, None)

    return fused


def kernel(src_token_ids, embedded_x, pe):
    B, L = src_token_ids.shape
    _, _, D = embedded_x.shape
    P = pe.shape[1]
    tokens = src_token_ids.reshape(L).astype(jnp.int32)
    emb2 = embedded_x.reshape(L, D)
    pe2 = pe.reshape(P, D)
    out2 = _make_fused_kernel(L, D, P)(tokens, emb2, pe2)
    return out2.reshape(B, L, D)


# speedup vs baseline: 1.0215x; 1.0215x over previous
import functools

import jax
import jax.numpy as jnp
from jax import lax
from jax.experimental import pallas as pl
from jax.experimental.pallas import tpu as pltpu
from jax.experimental.pallas import tpu_sc as plsc

NC = 2
NS = 16
NW = NC * NS
LANES = 16
VSIZE = 32768


def _mesh():
    return plsc.VectorSubcoreMesh(
        core_axis_name="c", subcore_axis_name="s",
        num_cores=NC, num_subcores=NS)


def _make_fused_kernel(L, D, P):
    nchunk = L // LANES
    b_per_w = L // NW
    K = 16
    nsub = b_per_w // K
    NR = 3
    NE = 2

    @functools.partial(
        pl.kernel,
        out_type=jax.ShapeDtypeStruct((L, D), jnp.float32),
        mesh=_mesh(),
        scratch_types=[
            pltpu.VMEM((VSIZE,), jnp.int32),
            pltpu.VMEM((L,), jnp.int32),
            pltpu.VMEM((L,), jnp.int32),
            pltpu.VMEM((L,), jnp.int32),
            pltpu.VMEM((L // NS,), jnp.int32),
            pltpu.VMEM((LANES,), jnp.int32),
            pltpu.VMEM((b_per_w,), jnp.int32),
            pltpu.VMEM((NR, K, D), jnp.float32),
            pltpu.VMEM((NE, K, D), jnp.float32),
            pltpu.VMEM_SHARED((L,), jnp.int32),
            pltpu.VMEM_SHARED((L,), jnp.int32),
            pltpu.SemaphoreType.DMA,
            pltpu.SemaphoreType.DMA,
            pltpu.SemaphoreType.DMA,
        ],
        compiler_params=pltpu.CompilerParams(needs_layout_passes=False),
    )
    def fused(tok_hbm, emb_hbm, pe_hbm, out_hbm,
              table, tok_v, pos_v, firstv_v, fbuf, shuf, idx_v,
              rows_v, emb_v, firstv_sh, pos_sh, gsem, esem, osem):
        sid = lax.axis_index("s")
        cid = lax.axis_index("c")
        wid = sid * NC + cid
        base = wid * b_per_w
        CPT = nchunk // NS
        TPT = CPT * LANES

        ce = [None] * nsub

        def issue_emb(j):
            ce[j] = pltpu.async_copy(
                emb_hbm.at[pl.ds(base + j * K, K)], emb_v.at[j % NE], esem)

        issue_emb(0)
        if nsub > 1:
            issue_emb(1)

        lane = lax.iota(jnp.int32, LANES)
        prev = jnp.maximum(lane - 1, 0)
        tstart = sid * TPT

        @pl.when(sid == 0)
        def _():
            pltpu.sync_copy(tok_hbm, tok_v)

        @pl.when(sid != 0)
        def _():
            pltpu.sync_copy(tok_hbm.at[pl.ds(tstart, TPT)],
                            tok_v.at[pl.ds(tstart, TPT)])

        def dedup_body(k, _):
            t = tok_v[pl.ds(tstart + k * LANES, LANES)]
            sk, sv = plsc.sort_key_val(t * LANES + lane, lane)
            shuf[...] = sk
            sp = plsc.load_gather(shuf, [prev])
            bound = ((sk >> 4) != (sp >> 4)) | (lane == 0)
            plsc.store_scatter(fbuf, [k * LANES + sv], bound.astype(jnp.int32))
            return 0

        with jax.named_scope("p1a_dedup"):
            lax.fori_loop(0, CPT, dedup_body, 0)
            pltpu.sync_copy(fbuf, firstv_sh.at[pl.ds(tstart, TPT)])

        @pl.when(sid == 0)
        def _():
            neg1 = jnp.full((LANES,), -1, jnp.int32)

            def init_body(i, _):
                table[pl.ds(i * LANES, LANES)] = neg1
                return 0

            lax.fori_loop(0, VSIZE // LANES, init_body, 0)

        plsc.subcore_barrier()

        @pl.when(sid == 0)
        def _():
          with jax.named_scope("p1b_rank"):
            pltpu.sync_copy(firstv_sh, firstv_v)

            def chunk_body(ci, counter):
                t = tok_v[pl.ds(ci * LANES, LANES)]
                g = plsc.load_gather(table, [t])
                firstv = firstv_v[pl.ds(ci * LANES, LANES)]
                new = (g < 0) & (firstv > 0)
                newi = new.astype(jnp.int32)
                inc = plsc.cumsum(newi)
                rank_new = counter + inc - newi
                plsc.store_scatter(table, [t], rank_new, mask=new)
                g2 = plsc.load_gather(table, [t])
                pos_v[pl.ds(ci * LANES, LANES)] = g2
                return counter + plsc.all_reduce_population_count(new)

            lax.fori_loop(0, nchunk, chunk_body, jnp.zeros((LANES,), jnp.int32))
            pltpu.sync_copy(pos_v, pos_sh)

        plsc.subcore_barrier()

        scope2 = jax.named_scope("p2_gather_add")
        scope2.__enter__()
        pltpu.sync_copy(pos_sh.at[pl.ds(base, b_per_w)], idx_v)
        cg = [None] * nsub
        co = [None] * nsub

        def issue_gather(j):
            cg[j] = pltpu.async_copy(
                pe_hbm.at[idx_v.at[pl.ds(j * K, K)]], rows_v.at[j % NR], gsem)

        issue_gather(0)
        if nsub > 1:
            issue_gather(1)
        for j in range(nsub):
            if j + 1 < nsub and j >= 1:
                issue_emb(j + 1)
            if j + 2 < nsub:
                if j + 2 >= NR:
                    co[j - 1].wait()
                issue_gather(j + 2)
            cg[j].wait()
            ce[j].wait()
            rs, es = j % NR, j % NE

            def add_body(r, _):
                for cc in range(D // LANES):
                    sl = pl.ds(cc * LANES, LANES)
                    rows_v[rs, r, sl] += emb_v[es, r, sl]
                return 0

            lax.fori_loop(0, K, add_body, 0)
            co[j] = pltpu.async_copy(
                rows_v.at[rs], out_hbm.at[pl.ds(base + j * K, K)], osem)
        for j in range(max(nsub - NR, 0), nsub):
            co[j].wait()
        scope2.__exit__(None, None, None)

    return fused


def kernel(src_token_ids, embedded_x, pe):
    B, L = src_token_ids.shape
    _, _, D = embedded_x.shape
    P = pe.shape[1]
    tokens = src_token_ids.reshape(L).astype(jnp.int32)
    emb2 = embedded_x.reshape(L, D)
    pe2 = pe.reshape(P, D)
    out2 = _make_fused_kernel(L, D, P)(tokens, emb2, pe2)
    return out2.reshape(B, L, D)


# speedup vs baseline: 1.2755x; 1.2486x over previous
import functools

import jax
import jax.numpy as jnp
from jax import lax
from jax.experimental import pallas as pl
from jax.experimental.pallas import tpu as pltpu
from jax.experimental.pallas import tpu_sc as plsc

NC = 2
NS = 16
NW = NC * NS
LANES = 16
VSIZE = 32768


def _mesh():
    return plsc.VectorSubcoreMesh(
        core_axis_name="c", subcore_axis_name="s",
        num_cores=NC, num_subcores=NS)


def _make_fused_kernel(L, D, P):
    nchunk = L // LANES
    b_per_w = L // NW
    K = 16
    nsub = b_per_w // K
    NR = 3
    NE = 2

    @functools.partial(
        pl.kernel,
        out_type=jax.ShapeDtypeStruct((L, D), jnp.float32),
        mesh=_mesh(),
        scratch_types=[
            pltpu.VMEM((VSIZE,), jnp.int32),
            pltpu.VMEM((L,), jnp.int32),
            pltpu.VMEM((L,), jnp.int32),
            pltpu.VMEM((L,), jnp.int32),
            pltpu.VMEM((L // NS,), jnp.int32),
            pltpu.VMEM((LANES,), jnp.int32),
            pltpu.VMEM((b_per_w,), jnp.int32),
            pltpu.VMEM((NR, K, D), jnp.float32),
            pltpu.VMEM((NE, K, D), jnp.float32),
            pltpu.VMEM_SHARED((L,), jnp.int32),
            pltpu.VMEM_SHARED((L,), jnp.int32),
            pltpu.SemaphoreType.DMA,
            pltpu.SemaphoreType.DMA,
            pltpu.SemaphoreType.DMA,
        ],
        compiler_params=pltpu.CompilerParams(needs_layout_passes=False),
    )
    def fused(tok_hbm, emb_hbm, pe_hbm, out_hbm,
              table, tok_v, pos_v, firstv_v, fbuf, shuf, idx_v,
              rows_v, emb_v, firstv_sh, pos_sh, gsem, esem, osem):
        sid = lax.axis_index("s")
        cid = lax.axis_index("c")
        wid = sid * NC + cid
        base = wid * b_per_w
        CPT = nchunk // NS
        TPT = CPT * LANES

        ce = [None] * nsub

        def issue_emb(j):
            ce[j] = pltpu.async_copy(
                emb_hbm.at[pl.ds(base + j * K, K)], emb_v.at[j % NE], esem)

        issue_emb(0)
        if nsub > 1:
            issue_emb(1)

        lane = lax.iota(jnp.int32, LANES)
        prev = jnp.maximum(lane - 1, 0)
        tstart = sid * TPT

        @pl.when(sid == 0)
        def _():
            pltpu.sync_copy(tok_hbm, tok_v)

        @pl.when(sid != 0)
        def _():
            pltpu.sync_copy(tok_hbm.at[pl.ds(tstart, TPT)],
                            tok_v.at[pl.ds(tstart, TPT)])

        def dedup_body(k, _):
            t = tok_v[pl.ds(tstart + k * LANES, LANES)]
            sk, sv = plsc.sort_key_val(t * LANES + lane, lane)
            shuf[...] = sk
            sp = plsc.load_gather(shuf, [prev])
            bound = ((sk >> 4) != (sp >> 4)) | (lane == 0)
            plsc.store_scatter(fbuf, [k * LANES + sv], bound.astype(jnp.int32))
            return 0

        with jax.named_scope("p1a_dedup"):
            lax.fori_loop(0, CPT, dedup_body, 0)
            pltpu.sync_copy(fbuf, firstv_sh.at[pl.ds(tstart, TPT)])

        @pl.when(sid == 0)
        def _():
            neg1 = jnp.full((LANES,), -1, jnp.int32)

            def init_body(i, _):
                table[pl.ds(i * LANES, LANES)] = neg1
                return 0

            lax.fori_loop(0, VSIZE // LANES, init_body, 0, unroll=8)

        plsc.subcore_barrier()

        @pl.when(sid == 0)
        def _():
          with jax.named_scope("p1b_rank"):
            pltpu.sync_copy(firstv_sh, firstv_v)

            def chunk_body(ci, counter):
                t = tok_v[pl.ds(ci * LANES, LANES)]
                g = plsc.load_gather(table, [t])
                firstv = firstv_v[pl.ds(ci * LANES, LANES)]
                new = (g < 0) & (firstv > 0)
                newi = new.astype(jnp.int32)
                inc = plsc.cumsum(newi)
                rank_new = counter + inc - newi
                plsc.store_scatter(table, [t], rank_new, mask=new)
                g2 = plsc.load_gather(table, [t])
                pos_v[pl.ds(ci * LANES, LANES)] = g2
                return counter + plsc.all_reduce_population_count(new)

            lax.fori_loop(0, nchunk, chunk_body, jnp.zeros((LANES,), jnp.int32))
            pltpu.sync_copy(pos_v, pos_sh)

        plsc.subcore_barrier()

        scope2 = jax.named_scope("p2_gather_add")
        scope2.__enter__()
        pltpu.sync_copy(pos_sh.at[pl.ds(base, b_per_w)], idx_v)
        cg = [None] * nsub
        co = [None] * nsub

        def issue_gather(j):
            cg[j] = pltpu.async_copy(
                pe_hbm.at[idx_v.at[pl.ds(j * K, K)]], rows_v.at[j % NR], gsem)

        issue_gather(0)
        if nsub > 1:
            issue_gather(1)
        for j in range(nsub):
            if j + 1 < nsub and j >= 1:
                issue_emb(j + 1)
            if j + 2 < nsub:
                if j + 2 >= NR:
                    co[j - 1].wait()
                issue_gather(j + 2)
            cg[j].wait()
            ce[j].wait()
            rs, es = j % NR, j % NE

            def add_body(r, _):
                for cc in range(D // LANES):
                    sl = pl.ds(cc * LANES, LANES)
                    plsc.addupdate(rows_v.at[rs, r, sl], emb_v[es, r, sl])
                return 0

            lax.fori_loop(0, K, add_body, 0)
            co[j] = pltpu.async_copy(
                rows_v.at[rs], out_hbm.at[pl.ds(base + j * K, K)], osem)
        for j in range(max(nsub - NR, 0), nsub):
            co[j].wait()
        scope2.__exit__(None, None, None)

    return fused


def kernel(src_token_ids, embedded_x, pe):
    B, L = src_token_ids.shape
    _, _, D = embedded_x.shape
    P = pe.shape[1]
    tokens = src_token_ids.reshape(L).astype(jnp.int32)
    emb2 = embedded_x.reshape(L, D)
    pe2 = pe.reshape(P, D)
    out2 = _make_fused_kernel(L, D, P)(tokens, emb2, pe2)
    return out2.reshape(B, L, D)


# speedup vs baseline: 1.3153x; 1.0312x over previous
import functools

import jax
import jax.numpy as jnp
from jax import lax
from jax.experimental import pallas as pl
from jax.experimental.pallas import tpu as pltpu
from jax.experimental.pallas import tpu_sc as plsc

NC = 2
NS = 16
NW = NC * NS
LANES = 16
VSIZE = 32768


def _mesh():
    return plsc.VectorSubcoreMesh(
        core_axis_name="c", subcore_axis_name="s",
        num_cores=NC, num_subcores=NS)


def _make_fused_kernel(L, D, P):
    nchunk = L // LANES
    b_per_w = L // NW
    K = 8
    nsub = b_per_w // K
    NR = 6
    NE = 4

    @functools.partial(
        pl.kernel,
        out_type=jax.ShapeDtypeStruct((L, D), jnp.float32),
        mesh=_mesh(),
        scratch_types=[
            pltpu.VMEM((VSIZE,), jnp.int32),
            pltpu.VMEM((L,), jnp.int32),
            pltpu.VMEM((L,), jnp.int32),
            pltpu.VMEM((L,), jnp.int32),
            pltpu.VMEM((L // NS,), jnp.int32),
            pltpu.VMEM((LANES,), jnp.int32),
            pltpu.VMEM((b_per_w,), jnp.int32),
            pltpu.VMEM((NR, K, D), jnp.float32),
            pltpu.VMEM((NE, K, D), jnp.float32),
            pltpu.VMEM_SHARED((L,), jnp.int32),
            pltpu.VMEM_SHARED((L,), jnp.int32),
            pltpu.SemaphoreType.DMA,
            pltpu.SemaphoreType.DMA,
            pltpu.SemaphoreType.DMA,
        ],
        compiler_params=pltpu.CompilerParams(needs_layout_passes=False),
    )
    def fused(tok_hbm, emb_hbm, pe_hbm, out_hbm,
              table, tok_v, pos_v, firstv_v, fbuf, shuf, idx_v,
              rows_v, emb_v, firstv_sh, pos_sh, gsem, esem, osem):
        sid = lax.axis_index("s")
        cid = lax.axis_index("c")
        wid = sid * NC + cid
        base = wid * b_per_w
        CPT = nchunk // NS
        TPT = CPT * LANES

        ce = [None] * nsub

        def issue_emb(j):
            ce[j] = pltpu.async_copy(
                emb_hbm.at[pl.ds(base + j * K, K)], emb_v.at[j % NE], esem)

        for j in range(min(NE, nsub)):
            issue_emb(j)

        lane = lax.iota(jnp.int32, LANES)
        prev = jnp.maximum(lane - 1, 0)
        tstart = sid * TPT

        @pl.when(sid == 0)
        def _():
            pltpu.sync_copy(tok_hbm, tok_v)

        @pl.when(sid != 0)
        def _():
            pltpu.sync_copy(tok_hbm.at[pl.ds(tstart, TPT)],
                            tok_v.at[pl.ds(tstart, TPT)])

        def dedup_body(k, _):
            t = tok_v[pl.ds(tstart + k * LANES, LANES)]
            sk, sv = plsc.sort_key_val(t * LANES + lane, lane)
            shuf[...] = sk
            sp = plsc.load_gather(shuf, [prev])
            bound = ((sk >> 4) != (sp >> 4)) | (lane == 0)
            plsc.store_scatter(fbuf, [k * LANES + sv], bound.astype(jnp.int32))
            return 0

        with jax.named_scope("p1a_dedup"):
            lax.fori_loop(0, CPT, dedup_body, 0)
            pltpu.sync_copy(fbuf, firstv_sh.at[pl.ds(tstart, TPT)])

        @pl.when(sid == 0)
        def _():
            neg1 = jnp.full((LANES,), -1, jnp.int32)

            def init_body(i, _):
                table[pl.ds(i * LANES, LANES)] = neg1
                return 0

            lax.fori_loop(0, VSIZE // LANES, init_body, 0, unroll=8)

        plsc.subcore_barrier()

        @pl.when(sid == 0)
        def _():
          with jax.named_scope("p1b_rank"):
            pltpu.sync_copy(firstv_sh, firstv_v)

            def chunk_body(ci, counter):
                t = tok_v[pl.ds(ci * LANES, LANES)]
                g = plsc.load_gather(table, [t])
                firstv = firstv_v[pl.ds(ci * LANES, LANES)]
                new = (g < 0) & (firstv > 0)
                newi = new.astype(jnp.int32)
                inc = plsc.cumsum(newi)
                rank_new = counter + inc - newi
                plsc.store_scatter(table, [t], rank_new, mask=new)
                g2 = plsc.load_gather(table, [t])
                pos_v[pl.ds(ci * LANES, LANES)] = g2
                return counter + plsc.all_reduce_population_count(new)

            lax.fori_loop(0, nchunk, chunk_body, jnp.zeros((LANES,), jnp.int32))
            pltpu.sync_copy(pos_v, pos_sh)

        plsc.subcore_barrier()

        scope2 = jax.named_scope("p2_gather_add")
        scope2.__enter__()
        pltpu.sync_copy(pos_sh.at[pl.ds(base, b_per_w)], idx_v)
        cg = [None] * nsub
        co = [None] * nsub

        def issue_gather(j):
            cg[j] = pltpu.async_copy(
                pe_hbm.at[idx_v.at[pl.ds(j * K, K)]], rows_v.at[j % NR], gsem)

        for j in range(min(3, nsub)):
            issue_gather(j)
        for j in range(nsub):
            if j + 2 < nsub and j >= NE - 2:
                issue_emb(j + 2)
            if j + 3 < nsub:
                if j + 3 >= NR:
                    co[j + 3 - NR].wait()
                issue_gather(j + 3)
            cg[j].wait()
            ce[j].wait()
            rs, es = j % NR, j % NE

            def add_body(r, _):
                for cc in range(D // LANES):
                    sl = pl.ds(cc * LANES, LANES)
                    plsc.addupdate(rows_v.at[rs, r, sl], emb_v[es, r, sl])
                return 0

            lax.fori_loop(0, K, add_body, 0)
            co[j] = pltpu.async_copy(
                rows_v.at[rs], out_hbm.at[pl.ds(base + j * K, K)], osem)
        for j in range(max(nsub - NR, 0), nsub):
            co[j].wait()
        scope2.__exit__(None, None, None)

    return fused


def kernel(src_token_ids, embedded_x, pe):
    B, L = src_token_ids.shape
    _, _, D = embedded_x.shape
    P = pe.shape[1]
    tokens = src_token_ids.reshape(L).astype(jnp.int32)
    emb2 = embedded_x.reshape(L, D)
    pe2 = pe.reshape(P, D)
    out2 = _make_fused_kernel(L, D, P)(tokens, emb2, pe2)
    return out2.reshape(B, L, D)


# speedup vs baseline: 1.4782x; 1.1238x over previous
import functools

import jax
import jax.numpy as jnp
from jax import lax
from jax.experimental import pallas as pl
from jax.experimental.pallas import tpu as pltpu
from jax.experimental.pallas import tpu_sc as plsc

NC = 2
NS = 16
NW = NC * NS
LANES = 16
VSIZE = 32768


def _mesh():
    return plsc.VectorSubcoreMesh(
        core_axis_name="c", subcore_axis_name="s",
        num_cores=NC, num_subcores=NS)


def _make_fused_kernel(L, D, P):
    nchunk = L // LANES
    b_per_w = L // NW
    K = 8
    nsub = b_per_w // K
    NR = 6
    NE = 4

    @functools.partial(
        pl.kernel,
        out_type=jax.ShapeDtypeStruct((L, D), jnp.float32),
        mesh=_mesh(),
        scratch_types=[
            pltpu.VMEM((VSIZE,), jnp.int32),
            pltpu.VMEM((L,), jnp.int32),
            pltpu.VMEM((L,), jnp.int32),
            pltpu.VMEM((L,), jnp.int32),
            pltpu.VMEM((L // NS,), jnp.int32),
            pltpu.VMEM((LANES,), jnp.int32),
            pltpu.VMEM((b_per_w,), jnp.int32),
            pltpu.VMEM((NR, K, D), jnp.float32),
            pltpu.VMEM((NE, K, D), jnp.float32),
            pltpu.VMEM_SHARED((L,), jnp.int32),
            pltpu.VMEM_SHARED((L,), jnp.int32),
            pltpu.SemaphoreType.DMA,
            pltpu.SemaphoreType.DMA,
            pltpu.SemaphoreType.DMA,
        ],
        compiler_params=pltpu.CompilerParams(needs_layout_passes=False),
    )
    def fused(tok_hbm, emb_hbm, pe_hbm, out_hbm,
              table, tok_v, pos_v, firstv_v, fbuf, shuf, idx_v,
              rows_v, emb_v, firstv_sh, pos_sh, gsem, esem, osem):
        sid = lax.axis_index("s")
        cid = lax.axis_index("c")
        wid = sid * NC + cid
        base = wid * b_per_w
        CPT = nchunk // NS
        TPT = CPT * LANES

        ce = [None] * nsub

        def issue_emb(j):
            ce[j] = pltpu.async_copy(
                emb_hbm.at[pl.ds(base + j * K, K)], emb_v.at[j % NE], esem)

        for j in range(min(NE, nsub)):
            issue_emb(j)

        lane = lax.iota(jnp.int32, LANES)
        prev = jnp.maximum(lane - 1, 0)
        tstart = sid * TPT

        @pl.when(sid == 0)
        def _():
            pltpu.sync_copy(tok_hbm, tok_v)

        @pl.when(sid != 0)
        def _():
            pltpu.sync_copy(tok_hbm.at[pl.ds(tstart, TPT)],
                            tok_v.at[pl.ds(tstart, TPT)])

        def dedup_body(k, _):
            t = tok_v[pl.ds(tstart + k * LANES, LANES)]
            sk, sv = plsc.sort_key_val(t * LANES + lane, lane)
            shuf[...] = sk
            sp = plsc.load_gather(shuf, [prev])
            bound = ((sk >> 4) != (sp >> 4)) | (lane == 0)
            plsc.store_scatter(fbuf, [k * LANES + sv], bound.astype(jnp.int32))
            return 0

        with jax.named_scope("p1a_dedup"):
            lax.fori_loop(0, CPT, dedup_body, 0)
            pltpu.sync_copy(fbuf, firstv_sh.at[pl.ds(tstart, TPT)])

        @pl.when(sid == 0)
        def _():
            neg1 = jnp.full((LANES,), -1, jnp.int32)

            def init_body(i, _):
                table[pl.ds(i * LANES, LANES)] = neg1
                return 0

            lax.fori_loop(0, VSIZE // LANES, init_body, 0, unroll=8)

        plsc.subcore_barrier()

        @pl.when(sid == 0)
        def _():
          with jax.named_scope("p1b_rank"):
            pltpu.sync_copy(firstv_sh, firstv_v)

            def chunk_body(ci, counter):
                t = tok_v[pl.ds(ci * LANES, LANES)]
                g = plsc.load_gather(table, [t])
                firstv = firstv_v[pl.ds(ci * LANES, LANES)]
                new = (g < 0) & (firstv > 0)
                newi = new.astype(jnp.int32)
                inc = plsc.cumsum(newi)
                rank_new = counter + inc - newi
                plsc.store_scatter(table, [t], rank_new, mask=new)
                g2 = plsc.load_gather(table, [t])
                pos_v[pl.ds(ci * LANES, LANES)] = g2
                return counter + plsc.all_reduce_population_count(new)

            lax.fori_loop(0, nchunk, chunk_body, jnp.zeros((LANES,), jnp.int32))
            pltpu.sync_copy(pos_v, pos_sh)

        plsc.subcore_barrier()

        scope2 = jax.named_scope("p2_gather_add")
        scope2.__enter__()
        pltpu.sync_copy(pos_sh.at[pl.ds(base, b_per_w)], idx_v)
        cg = [None] * nsub
        co = [None] * nsub

        def issue_gather(j):
            cg[j] = pltpu.async_copy(
                pe_hbm.at[idx_v.at[pl.ds(j * K, K)]], rows_v.at[j % NR], gsem)

        for j in range(min(3, nsub)):
            issue_gather(j)
        for j in range(nsub):
            if j + 2 < nsub and j >= NE - 2:
                issue_emb(j + 2)
            if j + 3 < nsub:
                if j + 3 >= NR:
                    co[j + 3 - NR].wait()
                issue_gather(j + 3)
            cg[j].wait()
            ce[j].wait()
            rs, es = j % NR, j % NE

            def add_body(cc, _):
                sl = pl.ds(cc * LANES, LANES)
                for r in range(K):
                    plsc.addupdate(rows_v.at[rs, r, sl], emb_v[es, r, sl])
                return 0

            lax.fori_loop(0, D // LANES, add_body, 0)
            co[j] = pltpu.async_copy(
                rows_v.at[rs], out_hbm.at[pl.ds(base + j * K, K)], osem)
        for j in range(max(nsub - NR, 0), nsub):
            co[j].wait()
        scope2.__exit__(None, None, None)

    return fused


def kernel(src_token_ids, embedded_x, pe):
    B, L = src_token_ids.shape
    _, _, D = embedded_x.shape
    P = pe.shape[1]
    tokens = src_token_ids.reshape(L).astype(jnp.int32)
    emb2 = embedded_x.reshape(L, D)
    pe2 = pe.reshape(P, D)
    out2 = _make_fused_kernel(L, D, P)(tokens, emb2, pe2)
    return out2.reshape(B, L, D)
